# Initial kernel scaffold; baseline (speedup 1.0000x reference)
import functools

import jax
import jax.numpy as jnp
from jax import lax
from jax.experimental import pallas as pl
from jax.experimental.pallas import tpu as pltpu
from jax.experimental.pallas import tpu_sc as plsc

N = 10000
E = 320000
F = 128
B = 1024
K = 128
NCHUNK = E // K
NP = 10112
NC = 2
NS = 16
NW = NC * NS
ROWS_PER_SUB = NP // NS

_mesh = plsc.VectorSubcoreMesh(core_axis_name="c", subcore_axis_name="s")


def _wid():
    return lax.axis_index("s") * NC + lax.axis_index("c")


def _chunk_trips(wid):
    return jnp.where(wid < NCHUNK % NW, NCHUNK // NW + 1, NCHUNK // NW)


@functools.partial(
    pl.kernel,
    out_type=jax.ShapeDtypeStruct((NC, 3, NP, 16), jnp.float32),
    mesh=_mesh,
    scratch_types=[
        pltpu.VMEM_SHARED((NP, 16), jnp.float32),
        pltpu.VMEM_SHARED((NP, 16), jnp.float32),
        pltpu.VMEM_SHARED((NP, 16), jnp.float32),
        pltpu.VMEM((K, 16), jnp.float32),
        pltpu.VMEM((K, 16), jnp.float32),
        pltpu.VMEM((K,), jnp.int32),
    ],
)
def _deg_kernel(dst3, zeros_hbm, ones_hbm, out, sh0, sh1, sh2, zb, ones_v, dstv):
    c = lax.axis_index("c")
    s = lax.axis_index("s")
    wid = _wid()
    shs = [sh0, sh1, sh2]
    pltpu.sync_copy(zeros_hbm.at[:, pl.ds(0, 16)], zb)
    pltpu.sync_copy(ones_hbm, ones_v)
    for r in range(3):
        for j in range(5):
            cz = s + NS * j

            @pl.when(cz < NP // K)
            def _():
                pltpu.sync_copy(zb, shs[r].at[pl.ds(cz * K, K)])

    plsc.subcore_barrier()
    trips = _chunk_trips(wid)

    def body(j, carry):
        ch = wid + NW * j
        for r in range(3):
            pltpu.sync_copy(dst3.at[r, pl.ds(ch * K, K)], dstv)
            pltpu.sync_copy(ones_v, shs[r].at[dstv], add=True)
        return carry

    lax.fori_loop(0, trips, body, 0)
    plsc.subcore_barrier()
    for r in range(3):
        pltpu.sync_copy(shs[r].at[pl.ds(s * ROWS_PER_SUB, ROWS_PER_SUB)],
                        out.at[c, r, pl.ds(s * ROWS_PER_SUB, ROWS_PER_SUB)])


@functools.partial(
    pl.kernel,
    out_type=jax.ShapeDtypeStruct((NC, 3, NP, F), jnp.float32),
    mesh=_mesh,
    scratch_types=[
        pltpu.VMEM_SHARED((NP, F), jnp.float32),
        pltpu.VMEM((K, F), jnp.float32),
        pltpu.VMEM((K, F), jnp.float32),
        pltpu.VMEM((K,), jnp.int32),
        pltpu.VMEM((K,), jnp.int32),
        pltpu.SemaphoreType.DMA,
    ],
)
def _scatter_kernel(hp0, hp1, hp2, src3, dst3, zeros_hbm, out,
                    acc, zb, rows, srcv, dstv, sem):
    c = lax.axis_index("c")
    s = lax.axis_index("s")
    wid = _wid()
    hps = [hp0, hp1, hp2]
    pltpu.sync_copy(zeros_hbm, zb)
    trips = _chunk_trips(wid)
    for r in range(3):
        for j in range(5):
            cz = s + NS * j

            @pl.when(cz < NP // K)
            def _():
                pltpu.sync_copy(zb, acc.at[pl.ds(cz * K, K)])

        plsc.subcore_barrier()

        def body(j, carry):
            ch = wid + NW * j
            pltpu.sync_copy(src3.at[r, pl.ds(ch * K, K)], srcv)
            pltpu.sync_copy(dst3.at[r, pl.ds(ch * K, K)], dstv)
            pltpu.async_copy(hps[r].at[srcv], rows, sem).wait()
            pltpu.sync_copy(rows, acc.at[dstv], add=True)
            return carry

        lax.fori_loop(0, trips, body, 0)
        plsc.subcore_barrier()
        pltpu.sync_copy(acc.at[pl.ds(s * ROWS_PER_SUB, ROWS_PER_SUB)],
                        out.at[c, r, pl.ds(s * ROWS_PER_SUB, ROWS_PER_SUB)])
        if r < 2:
            plsc.subcore_barrier()


@functools.partial(
    pl.kernel,
    out_type=jax.ShapeDtypeStruct((B, F), jnp.float32),
    mesh=_mesh,
    scratch_types=[
        pltpu.VMEM((B // NW,), jnp.int32),
        pltpu.VMEM((B // NW, F), jnp.float32),
        pltpu.SemaphoreType.DMA,
    ],
)
def _gather_kernel(g_hbm, idx_hbm, out, idx_v, rows_v, sem):
    wid = _wid()
    base = wid * (B // NW)
    pltpu.sync_copy(idx_hbm.at[pl.ds(base, B // NW)], idx_v)
    pltpu.async_copy(g_hbm.at[idx_v], rows_v, sem).wait()
    pltpu.sync_copy(rows_v, out.at[pl.ds(base, B // NW)])


_BR = 1000


def _c1_body(x, d0a, d0b, d1a, d1b, d2a, d2b, W0, W1, W2,
             hp0, hp1, hp2, dv0, dv1, dv2):
    degs = [d0a, d0b, d1a, d1b, d2a, d2b]
    Ws = [W0, W1, W2]
    hps = [hp0, hp1, hp2]
    dvs = [dv0, dv1, dv2]
    xb = x[...]
    for r in range(3):
        deg = degs[2 * r][0, 0, :, 0:1] + degs[2 * r + 1][0, 0, :, 0:1] + 1.0
        dinv = lax.rsqrt(deg)
        h = jnp.dot(xb, Ws[r][...], preferred_element_type=jnp.float32)
        hps[r][...] = h * dinv
        dvs[r][...] = dinv


def _c1(x, degp, W1s):
    spec_x = pl.BlockSpec((_BR, F), lambda i: (i, 0))
    spec_deg = [pl.BlockSpec((1, 1, _BR, 16),
                             (lambda i, c=c, r=r: (c, r, i, 0)))
                for r in range(3) for c in range(2)]
    spec_w = pl.BlockSpec((F, F), lambda i: (0, 0))
    spec_o = pl.BlockSpec((_BR, F), lambda i: (i, 0))
    spec_dv = pl.BlockSpec((_BR, 1), lambda i: (i, 0))
    return pl.pallas_call(
        _c1_body,
        grid=(N // _BR,),
        in_specs=[spec_x] + spec_deg + [spec_w] * 3,
        out_specs=[spec_o] * 3 + [spec_dv] * 3,
        out_shape=[jax.ShapeDtypeStruct((N, F), jnp.float32)] * 3
        + [jax.ShapeDtypeStruct((N, 1), jnp.float32)] * 3,
    )(x, degp, degp, degp, degp, degp, degp, *W1s)


def _ec_body(a00, a01, a10, a11, a20, a21, hp0, hp1, hp2,
             dv0, dv1, dv2, b0, b1, b2, W0, W1, W2, o0, o1, o2):
    accs = [(a00, a01), (a10, a11), (a20, a21)]
    hps = [hp0, hp1, hp2]
    dvs = [dv0, dv1, dv2]
    bs = [b0, b1, b2]
    Ws = [W0, W1, W2]
    outs = [o0, o1, o2]
    h = None
    for r in range(3):
        acc = accs[r][0][0, 0] + accs[r][1][0, 0]
        full = (acc + hps[r][...]) * dvs[r][...] + bs[r][...]
        v = jnp.maximum(full, 0.0)
        h = v if h is None else jnp.maximum(h, v)
    for r in range(3):
        o = jnp.dot(h, Ws[r][...], preferred_element_type=jnp.float32)
        outs[r][...] = o * dvs[r][...]


def _acc_specs():
    return [pl.BlockSpec((1, 1, _BR, F),
                         (lambda i, c=c, r=r: (c, r, i, 0)))
            for r in range(3) for c in range(2)]


def _ec(accp, hps, dvs, bs, W2s):
    spec_h = pl.BlockSpec((_BR, F), lambda i: (i, 0))
    spec_dv = pl.BlockSpec((_BR, 1), lambda i: (i, 0))
    spec_b = pl.BlockSpec((1, F), lambda i: (0, 0))
    spec_w = pl.BlockSpec((F, F), lambda i: (0, 0))
    return pl.pallas_call(
        _ec_body,
        grid=(N // _BR,),
        in_specs=_acc_specs() + [spec_h] * 3 + [spec_dv] * 3
        + [spec_b] * 3 + [spec_w] * 3,
        out_specs=[spec_h] * 3,
        out_shape=[jax.ShapeDtypeStruct((N, F), jnp.float32)] * 3,
    )(accp, accp, accp, accp, accp, accp, *hps, *dvs, *bs, *W2s)


def _e2_body(a00, a01, a10, a11, a20, a21, hp0, hp1, hp2,
             dv0, dv1, dv2, b0, b1, b2, g):
    accs = [(a00, a01), (a10, a11), (a20, a21)]
    hps = [hp0, hp1, hp2]
    dvs = [dv0, dv1, dv2]
    bs = [b0, b1, b2]
    h = None
    for r in range(3):
        acc = accs[r][0][0, 0] + accs[r][1][0, 0]
        full = (acc + hps[r][...]) * dvs[r][...] + bs[r][...]
        v = jnp.maximum(full, 0.0)
        h = v if h is None else jnp.maximum(h, v)
    g[...] = h


def _e2(accp, hps, dvs, bs):
    spec_h = pl.BlockSpec((_BR, F), lambda i: (i, 0))
    spec_dv = pl.BlockSpec((_BR, 1), lambda i: (i, 0))
    spec_b = pl.BlockSpec((1, F), lambda i: (0, 0))
    return pl.pallas_call(
        _e2_body,
        grid=(N // _BR,),
        in_specs=_acc_specs() + [spec_h] * 3 + [spec_dv] * 3 + [spec_b] * 3,
        out_specs=spec_h,
        out_shape=jax.ShapeDtypeStruct((N, F), jnp.float32),
    )(accp, accp, accp, accp, accp, accp, *hps, *dvs, *bs)


def kernel(x, syntactic_edge_index, sequential_edge_index, semantic_edge_index,
           index, W1_syn, b1_syn, W1_seq, b1_seq, W1_sem, b1_sem,
           W2_syn, b2_syn, W2_seq, b2_seq, W2_sem, b2_sem):
    es = [syntactic_edge_index, sequential_edge_index, semantic_edge_index]
    src3 = jnp.stack([e[0] for e in es])
    dst3 = jnp.stack([e[1] for e in es])
    zeros128 = jnp.zeros((K, F), jnp.float32)
    ones16 = jnp.ones((K, 16), jnp.float32)

    degp = _deg_kernel(dst3, zeros128, ones16)

    W1s = [W1_syn, W1_seq, W1_sem]
    b1s = [b1_syn.reshape(1, F), b1_seq.reshape(1, F), b1_sem.reshape(1, F)]
    W2s = [W2_syn, W2_seq, W2_sem]
    b2s = [b2_syn.reshape(1, F), b2_seq.reshape(1, F), b2_sem.reshape(1, F)]

    hp0, hp1, hp2, dv0, dv1, dv2 = _c1(x, degp, W1s)

    acc1 = _scatter_kernel(hp0, hp1, hp2, src3, dst3, zeros128)

    h2p = _ec(acc1, [hp0, hp1, hp2], [dv0, dv1, dv2], b1s, W2s)

    acc2 = _scatter_kernel(h2p[0], h2p[1], h2p[2], src3, dst3, zeros128)

    g = _e2(acc2, h2p, [dv0, dv1, dv2], b2s)
    return _gather_kernel(g, index)


# speedup vs baseline: 13.2278x; 13.2278x over previous
import functools

import jax
import jax.numpy as jnp
from jax import lax
from jax.experimental import pallas as pl
from jax.experimental.pallas import tpu as pltpu
from jax.experimental.pallas import tpu_sc as plsc

N = 10000
E = 320000
F = 128
B = 1024
K = 128
NCHUNK = E // K
NP = 10112
NC = 2
NS = 16
NW = NC * NS
ROWS_PER_SUB = NP // NS

_mesh = plsc.VectorSubcoreMesh(core_axis_name="c", subcore_axis_name="s")


def _wid():
    return lax.axis_index("s") * NC + lax.axis_index("c")


def _chunk_trips(wid):
    return jnp.where(wid < NCHUNK % NW, NCHUNK // NW + 1, NCHUNK // NW)


@functools.partial(
    pl.kernel,
    out_type=jax.ShapeDtypeStruct((NC, NP, F), jnp.float32),
    mesh=_mesh,
    scratch_types=[
        pltpu.VMEM_SHARED((NP, F), jnp.float32),
        pltpu.VMEM((K, F), jnp.float32),
        pltpu.VMEM((K,), jnp.int32),
    ],
)
def _deg_kernel(dsta, dstb, dstc, zeros_hbm, m0, m1, m2, out,
                shd, mv, dstv):
    c = lax.axis_index("c")
    s = lax.axis_index("s")
    wid = _wid()
    dsts = [dsta, dstb, dstc]
    ms = [m0, m1, m2]
    for j in range(5):
        cz = s + NS * j

        @pl.when(cz < NP // K)
        def _():
            pltpu.sync_copy(zeros_hbm, shd.at[pl.ds(cz * K, K)])

    plsc.subcore_barrier()
    trips = _chunk_trips(wid)
    for r in range(3):
        pltpu.sync_copy(ms[r], mv)

        def body(j, carry):
            ch = wid + NW * j
            pltpu.sync_copy(dsts[r].at[pl.ds(ch * K, K)], dstv)
            pltpu.sync_copy(mv, shd.at[dstv], add=True)
            return carry

        lax.fori_loop(0, trips, body, 0)
    plsc.subcore_barrier()
    pltpu.sync_copy(shd.at[pl.ds(s * ROWS_PER_SUB, ROWS_PER_SUB)],
                    out.at[c, pl.ds(s * ROWS_PER_SUB, ROWS_PER_SUB)])


@functools.partial(
    pl.kernel,
    out_type=jax.ShapeDtypeStruct((NC, 3, NP, F), jnp.float32),
    mesh=_mesh,
    scratch_types=[
        pltpu.VMEM_SHARED((NP, F), jnp.float32),
        pltpu.VMEM((K, F), jnp.float32),
        pltpu.VMEM((K,), jnp.int32),
        pltpu.VMEM((K,), jnp.int32),
        pltpu.SemaphoreType.DMA,
    ],
)
def _scatter_kernel(hp0, hp1, hp2, srca, srcb, srcc, dsta, dstb, dstc,
                    zeros_hbm, out, acc, rows, srcv, dstv, sem):
    c = lax.axis_index("c")
    s = lax.axis_index("s")
    wid = _wid()
    hps = [hp0, hp1, hp2]
    srcs = [srca, srcb, srcc]
    dsts = [dsta, dstb, dstc]
    trips = _chunk_trips(wid)
    for r in range(3):
        for j in range(5):
            cz = s + NS * j

            @pl.when(cz < NP // K)
            def _():
                pltpu.sync_copy(zeros_hbm, acc.at[pl.ds(cz * K, K)])

        plsc.subcore_barrier()

        def body(j, carry):
            ch = wid + NW * j
            pltpu.sync_copy(srcs[r].at[pl.ds(ch * K, K)], srcv)
            pltpu.sync_copy(dsts[r].at[pl.ds(ch * K, K)], dstv)
            pltpu.async_copy(hps[r].at[srcv], rows, sem).wait()
            pltpu.sync_copy(rows, acc.at[dstv], add=True)
            return carry

        lax.fori_loop(0, trips, body, 0)
        plsc.subcore_barrier()
        pltpu.sync_copy(acc.at[pl.ds(s * ROWS_PER_SUB, ROWS_PER_SUB)],
                        out.at[c, r, pl.ds(s * ROWS_PER_SUB, ROWS_PER_SUB)])
        if r < 2:
            plsc.subcore_barrier()


@functools.partial(
    pl.kernel,
    out_type=jax.ShapeDtypeStruct((B, F), jnp.float32),
    mesh=_mesh,
    scratch_types=[
        pltpu.VMEM((B // NW,), jnp.int32),
        pltpu.VMEM((B // NW, F), jnp.float32),
        pltpu.SemaphoreType.DMA,
    ],
)
def _gather_kernel(g_hbm, idx_hbm, out, idx_v, rows_v, sem):
    wid = _wid()
    base = wid * (B // NW)
    pltpu.sync_copy(idx_hbm.at[pl.ds(base, B // NW)], idx_v)
    pltpu.async_copy(g_hbm.at[idx_v], rows_v, sem).wait()
    pltpu.sync_copy(rows_v, out.at[pl.ds(base, B // NW)])


_BR = 1000


def _c1_body(x, dga, dgb, W0, W1, W2,
             hp0, hp1, hp2, dv0, dv1, dv2):
    Ws = [W0, W1, W2]
    hps = [hp0, hp1, hp2]
    dvs = [dv0, dv1, dv2]
    xb = x[...]
    for r in range(3):
        deg = dga[0, :, r:r + 1] + dgb[0, :, r:r + 1] + 1.0
        dinv = lax.rsqrt(deg)
        h = jnp.dot(xb, Ws[r][...], preferred_element_type=jnp.float32)
        hps[r][...] = h * dinv
        dvs[r][...] = dinv


def _c1(x, degp, W1s):
    spec_x = pl.BlockSpec((_BR, F), lambda i: (i, 0))
    spec_deg = [pl.BlockSpec((1, _BR, F), (lambda i, c=c: (c, i, 0)))
                for c in range(2)]
    spec_w = pl.BlockSpec((F, F), lambda i: (0, 0))
    spec_o = pl.BlockSpec((_BR, F), lambda i: (i, 0))
    spec_dv = pl.BlockSpec((_BR, 1), lambda i: (i, 0))
    return pl.pallas_call(
        _c1_body,
        grid=(N // _BR,),
        in_specs=[spec_x] + spec_deg + [spec_w] * 3,
        out_specs=[spec_o] * 3 + [spec_dv] * 3,
        out_shape=[jax.ShapeDtypeStruct((N, F), jnp.float32)] * 3
        + [jax.ShapeDtypeStruct((N, 1), jnp.float32)] * 3,
    )(x, degp, degp, *W1s)


def _ec_body(a00, a01, a10, a11, a20, a21, hp0, hp1, hp2,
             dv0, dv1, dv2, b0, b1, b2, W0, W1, W2, o0, o1, o2):
    accs = [(a00, a01), (a10, a11), (a20, a21)]
    hps = [hp0, hp1, hp2]
    dvs = [dv0, dv1, dv2]
    bs = [b0, b1, b2]
    Ws = [W0, W1, W2]
    outs = [o0, o1, o2]
    h = None
    for r in range(3):
        acc = accs[r][0][0, 0] + accs[r][1][0, 0]
        full = (acc + hps[r][...]) * dvs[r][...] + bs[r][...]
        v = jnp.maximum(full, 0.0)
        h = v if h is None else jnp.maximum(h, v)
    for r in range(3):
        o = jnp.dot(h, Ws[r][...], preferred_element_type=jnp.float32)
        outs[r][...] = o * dvs[r][...]


def _acc_specs():
    return [pl.BlockSpec((1, 1, _BR, F),
                         (lambda i, c=c, r=r: (c, r, i, 0)))
            for r in range(3) for c in range(2)]


def _ec(accp, hps, dvs, bs, W2s):
    spec_h = pl.BlockSpec((_BR, F), lambda i: (i, 0))
    spec_dv = pl.BlockSpec((_BR, 1), lambda i: (i, 0))
    spec_b = pl.BlockSpec((1, F), lambda i: (0, 0))
    spec_w = pl.BlockSpec((F, F), lambda i: (0, 0))
    return pl.pallas_call(
        _ec_body,
        grid=(N // _BR,),
        in_specs=_acc_specs() + [spec_h] * 3 + [spec_dv] * 3
        + [spec_b] * 3 + [spec_w] * 3,
        out_specs=[spec_h] * 3,
        out_shape=[jax.ShapeDtypeStruct((N, F), jnp.float32)] * 3,
    )(accp, accp, accp, accp, accp, accp, *hps, *dvs, *bs, *W2s)


def _e2_body(a00, a01, a10, a11, a20, a21, hp0, hp1, hp2,
             dv0, dv1, dv2, b0, b1, b2, g):
    accs = [(a00, a01), (a10, a11), (a20, a21)]
    hps = [hp0, hp1, hp2]
    dvs = [dv0, dv1, dv2]
    bs = [b0, b1, b2]
    h = None
    for r in range(3):
        acc = accs[r][0][0, 0] + accs[r][1][0, 0]
        full = (acc + hps[r][...]) * dvs[r][...] + bs[r][...]
        v = jnp.maximum(full, 0.0)
        h = v if h is None else jnp.maximum(h, v)
    g[...] = h


def _e2(accp, hps, dvs, bs):
    spec_h = pl.BlockSpec((_BR, F), lambda i: (i, 0))
    spec_dv = pl.BlockSpec((_BR, 1), lambda i: (i, 0))
    spec_b = pl.BlockSpec((1, F), lambda i: (0, 0))
    return pl.pallas_call(
        _e2_body,
        grid=(N // _BR,),
        in_specs=_acc_specs() + [spec_h] * 3 + [spec_dv] * 3 + [spec_b] * 3,
        out_specs=spec_h,
        out_shape=jax.ShapeDtypeStruct((N, F), jnp.float32),
    )(accp, accp, accp, accp, accp, accp, *hps, *dvs, *bs)


def kernel(x, syntactic_edge_index, sequential_edge_index, semantic_edge_index,
           index, W1_syn, b1_syn, W1_seq, b1_seq, W1_sem, b1_sem,
           W2_syn, b2_syn, W2_seq, b2_seq, W2_sem, b2_sem):
    es = [syntactic_edge_index, sequential_edge_index, semantic_edge_index]
    srcs = [e[0] for e in es]
    dsts = [e[1] for e in es]
    zeros128 = jnp.zeros((K, F), jnp.float32)
    lane = lax.broadcasted_iota(jnp.int32, (K, F), 1)
    masks = [(lane == r).astype(jnp.float32) for r in range(3)]

    degp = _deg_kernel(dsts[0], dsts[1], dsts[2], zeros128, *masks)

    W1s = [W1_syn, W1_seq, W1_sem]
    b1s = [b1_syn.reshape(1, F), b1_seq.reshape(1, F), b1_sem.reshape(1, F)]
    W2s = [W2_syn, W2_seq, W2_sem]
    b2s = [b2_syn.reshape(1, F), b2_seq.reshape(1, F), b2_sem.reshape(1, F)]

    hp0, hp1, hp2, dv0, dv1, dv2 = _c1(x, degp, W1s)

    acc1 = _scatter_kernel(hp0, hp1, hp2, *srcs, *dsts, zeros128)

    h2p = _ec(acc1, [hp0, hp1, hp2], [dv0, dv1, dv2], b1s, W2s)

    acc2 = _scatter_kernel(h2p[0], h2p[1], h2p[2], *srcs, *dsts, zeros128)

    g = _e2(acc2, h2p, [dv0, dv1, dv2], b2s)
    return _gather_kernel(g, index)
